# Initial kernel scaffold; baseline (speedup 1.0000x reference)
import jax
import jax.numpy as jnp
from jax.experimental import pallas as pl


def kernel(query_features, reference_points, W_val, b_val, W_off, b_off, W_att, b_att, W_out, b_out):
    raise NotImplementedError("write your pallas kernel here")


# speedup vs baseline: 22.6097x; 22.6097x over previous
import jax
import jax.numpy as jnp
from jax.experimental import pallas as pl

N, L, E = 2, 1024, 256
H, P, K = 8, 4, 4
D = E // H
NH = N * H
LB = 256
INF = 3.0e38


def _matmul_bias_kernel(x_ref, w_ref, b_ref, o_ref):
    o_ref[...] = (
        jnp.dot(x_ref[...], w_ref[...], preferred_element_type=jnp.float32)
        + b_ref[...]
    )


def _matmul_bias(x, w, b):
    m, _ = x.shape
    n = w.shape[1]
    return pl.pallas_call(
        _matmul_bias_kernel,
        out_shape=jax.ShapeDtypeStruct((m, n), jnp.float32),
    )(x, w, b.reshape(1, n))


def _deform_kernel(off_ref, rpq_ref, rpkT_ref, val_ref, att_ref, o_ref):
    rpkT = rpkT_ref[0]
    rpk2 = (rpkT[0:1, :] * rpkT[0:1, :]
            + rpkT[1:2, :] * rpkT[1:2, :]
            + rpkT[2:3, :] * rpkT[2:3, :])

    a_cols = [att_ref[0, :, p:p + 1] for p in range(P)]
    amax = jnp.maximum(jnp.maximum(a_cols[0], a_cols[1]),
                       jnp.maximum(a_cols[2], a_cols[3]))
    e_cols = [jnp.exp(a - amax) for a in a_cols]
    esum = e_cols[0] + e_cols[1] + e_cols[2] + e_cols[3]
    att_cols = [e / esum for e in e_cols]

    iota = jax.lax.broadcasted_iota(jnp.int32, (LB, L), 1)
    cmat = jnp.zeros((LB, L), jnp.float32)
    for p in range(P):
        s2 = jnp.zeros((LB, 1), jnp.float32)
        cross = jnp.zeros((LB, L), jnp.float32)
        for c in range(3):
            sc = rpq_ref[0, :, c:c + 1] + off_ref[0, p, :, c:c + 1]
            s2 = s2 + sc * sc
            sc_b = sc.astype(jnp.bfloat16).astype(jnp.float32)
            rk_b = rpkT[c:c + 1, :].astype(jnp.bfloat16).astype(jnp.float32)
            cross = cross + sc_b * rk_b
        d2 = jnp.maximum(s2 + rpk2 - 2.0 * cross, 0.0)

        acc = jnp.zeros((LB, L), jnp.float32)
        ssum = jnp.zeros((LB, 1), jnp.float32)
        for _ in range(K):
            rowmin = jnp.min(d2, axis=-1, keepdims=True)
            cand = jnp.where(d2 == rowmin, iota, jnp.int32(L))
            idx = jnp.min(cand, axis=-1, keepdims=True)
            sel = iota == idx
            w = 1.0 / (jnp.sqrt(rowmin) + 1e-8)
            acc = jnp.where(sel, w, acc)
            ssum = ssum + w
            d2 = jnp.where(sel, INF, d2)
        cmat = cmat + acc * (att_cols[p] / ssum)

    o_ref[0] = jnp.dot(cmat, val_ref[0], preferred_element_type=jnp.float32)


def kernel(query_features, reference_points, W_val, b_val, W_off, b_off,
           W_att, b_att, W_out, b_out):
    qf = query_features.reshape(N * L, E)
    Wc = jnp.concatenate([W_val, W_off, W_att], axis=1)
    bc = jnp.concatenate([b_val, b_off, b_att])
    proj = _matmul_bias(qf, Wc, bc)

    pv = proj[:, :E].reshape(N, L, H, D)
    VAL = pv.transpose(2, 0, 1, 3).reshape(NH, L, D)
    po = proj[:, E:E + H * P * 3].reshape(N, L, H, P, 3)
    OFF = po.transpose(0, 2, 3, 1, 4).reshape(NH, P, L, 3)
    pa = proj[:, E + H * P * 3:].reshape(N, L, H, P)
    ATT = pa.transpose(0, 2, 1, 3).reshape(NH, L, P)

    rpT = reference_points.transpose(0, 2, 1)

    heads = pl.pallas_call(
        _deform_kernel,
        grid=(NH, L // LB),
        in_specs=[
            pl.BlockSpec((1, P, LB, 3), lambda i, j: (i, 0, j, 0)),
            pl.BlockSpec((1, LB, 3), lambda i, j: (i // H, j, 0)),
            pl.BlockSpec((1, 3, L), lambda i, j: (i % N, 0, 0)),
            pl.BlockSpec((1, L, D), lambda i, j: (i, 0, 0)),
            pl.BlockSpec((1, LB, P), lambda i, j: (i, j, 0)),
        ],
        out_specs=pl.BlockSpec((1, LB, D), lambda i, j: (i, j, 0)),
        out_shape=jax.ShapeDtypeStruct((NH, L, D), jnp.float32),
    )(OFF, reference_points, rpT, VAL, ATT)

    out_flat = heads.reshape(N, H, L, D).transpose(0, 2, 1, 3).reshape(N * L, E)
    out = _matmul_bias(out_flat, W_out, b_out)
    return out.reshape(N, L, E)


# speedup vs baseline: 25.9859x; 1.1493x over previous
import jax
import jax.numpy as jnp
from jax.experimental import pallas as pl

N, L, E = 2, 1024, 256
H, P, K = 8, 4, 4
D = E // H
NH = N * H
LB = 256
INF = 3.0e38


def _matmul_bias_kernel(x_ref, w_ref, b_ref, o_ref):
    o_ref[...] = (
        jnp.dot(x_ref[...], w_ref[...], preferred_element_type=jnp.float32)
        + b_ref[...]
    )


def _matmul_bias(x, w, b):
    m, _ = x.shape
    n = w.shape[1]
    return pl.pallas_call(
        _matmul_bias_kernel,
        out_shape=jax.ShapeDtypeStruct((m, n), jnp.float32),
    )(x, w, b.reshape(1, n))


def _deform_kernel(off_ref, rpq_ref, rpkT_ref, val_ref, att_ref, o_ref):
    M = P * LB
    rpkT = rpkT_ref[0]
    rpk2 = (rpkT[0:1, :] * rpkT[0:1, :]
            + rpkT[1:2, :] * rpkT[1:2, :]
            + rpkT[2:3, :] * rpkT[2:3, :])

    a_cols = [att_ref[0, :, p:p + 1] for p in range(P)]
    amax = jnp.maximum(jnp.maximum(a_cols[0], a_cols[1]),
                       jnp.maximum(a_cols[2], a_cols[3]))
    e_cols = [jnp.exp(a - amax) for a in a_cols]
    esum = e_cols[0] + e_cols[1] + e_cols[2] + e_cols[3]
    att4 = jnp.concatenate([e / esum for e in e_cols], axis=0)

    rpq = rpq_ref[0]
    samp = jnp.concatenate(
        [rpq + off_ref[0, p] for p in range(P)], axis=0)
    s2 = (samp[:, 0:1] * samp[:, 0:1]
          + samp[:, 1:2] * samp[:, 1:2]
          + samp[:, 2:3] * samp[:, 2:3])
    cross = jax.lax.dot_general(
        samp.astype(jnp.bfloat16), rpkT.astype(jnp.bfloat16),
        (((1,), (0,)), ((), ())),
        preferred_element_type=jnp.float32)
    d2 = jnp.maximum(s2 + rpk2 - 2.0 * cross, 0.0)

    iota = jax.lax.broadcasted_iota(jnp.int32, (M, L), 1)
    acc = jnp.zeros((M, L), jnp.float32)
    ssum = jnp.zeros((M, 1), jnp.float32)
    for _ in range(K):
        rowmin = jnp.min(d2, axis=-1, keepdims=True)
        cand = jnp.where(d2 == rowmin, iota, jnp.int32(L))
        idx = jnp.min(cand, axis=-1, keepdims=True)
        sel = iota == idx
        w = 1.0 / (jnp.sqrt(rowmin) + 1e-8)
        acc = jnp.where(sel, w, acc)
        ssum = ssum + w
        d2 = jnp.where(sel, INF, d2)
    wmat = acc * (att4 / ssum)

    out4 = jnp.dot(wmat, val_ref[0], preferred_element_type=jnp.float32)
    o_ref[0] = (out4[0 * LB:1 * LB] + out4[1 * LB:2 * LB]
                + out4[2 * LB:3 * LB] + out4[3 * LB:4 * LB])


def kernel(query_features, reference_points, W_val, b_val, W_off, b_off,
           W_att, b_att, W_out, b_out):
    qf = query_features.reshape(N * L, E)
    Wc = jnp.concatenate([W_val, W_off, W_att], axis=1)
    bc = jnp.concatenate([b_val, b_off, b_att])
    proj = _matmul_bias(qf, Wc, bc)

    pv = proj[:, :E].reshape(N, L, H, D)
    VAL = pv.transpose(2, 0, 1, 3).reshape(NH, L, D)
    po = proj[:, E:E + H * P * 3].reshape(N, L, H, P, 3)
    OFF = po.transpose(0, 2, 3, 1, 4).reshape(NH, P, L, 3)
    pa = proj[:, E + H * P * 3:].reshape(N, L, H, P)
    ATT = pa.transpose(0, 2, 1, 3).reshape(NH, L, P)

    rpT = reference_points.transpose(0, 2, 1)

    heads = pl.pallas_call(
        _deform_kernel,
        grid=(NH, L // LB),
        in_specs=[
            pl.BlockSpec((1, P, LB, 3), lambda i, j: (i, 0, j, 0)),
            pl.BlockSpec((1, LB, 3), lambda i, j: (i // H, j, 0)),
            pl.BlockSpec((1, 3, L), lambda i, j: (i % N, 0, 0)),
            pl.BlockSpec((1, L, D), lambda i, j: (i, 0, 0)),
            pl.BlockSpec((1, LB, P), lambda i, j: (i, j, 0)),
        ],
        out_specs=pl.BlockSpec((1, LB, D), lambda i, j: (i, j, 0)),
        out_shape=jax.ShapeDtypeStruct((NH, L, D), jnp.float32),
    )(OFF, reference_points, rpT, VAL, ATT)

    out_flat = heads.reshape(N, H, L, D).transpose(0, 2, 1, 3).reshape(N * L, E)
    out = _matmul_bias(out_flat, W_out, b_out)
    return out.reshape(N, L, E)


# speedup vs baseline: 30.3650x; 1.1685x over previous
import jax
import jax.numpy as jnp
from jax.experimental import pallas as pl
from jax.experimental.pallas import tpu as pltpu

N, L, E = 2, 1024, 256
H, P, K = 8, 4, 4
D = E // H
NH = N * H
LB = 256
OA = P * 3 + P
INF = 3.0e38


def _proj_kernel(qf_ref, wc_ref, bc_ref, valh_ref, oah_ref):
    x = (jnp.dot(qf_ref[...], wc_ref[...], preferred_element_type=jnp.float32)
         + bc_ref[...])
    for h in range(H):
        valh_ref[h] = x[:, h * D:(h + 1) * D]
        oah_ref[h, :, 0:P * 3] = x[:, E + h * P * 3:E + (h + 1) * P * 3]
        oah_ref[h, :, P * 3:OA] = x[:, E + H * P * 3 + h * P:
                                    E + H * P * 3 + (h + 1) * P]


def _deform_kernel(oa_ref, rpq_ref, rpkT_ref, val_ref, o_ref):
    M = P * LB
    rpkT = rpkT_ref[0]
    rpk2 = (rpkT[0:1, :] * rpkT[0:1, :]
            + rpkT[1:2, :] * rpkT[1:2, :]
            + rpkT[2:3, :] * rpkT[2:3, :])

    a_cols = [oa_ref[0, :, P * 3 + p:P * 3 + p + 1] for p in range(P)]
    amax = jnp.maximum(jnp.maximum(a_cols[0], a_cols[1]),
                       jnp.maximum(a_cols[2], a_cols[3]))
    e_cols = [jnp.exp(a - amax) for a in a_cols]
    esum = e_cols[0] + e_cols[1] + e_cols[2] + e_cols[3]
    att4 = jnp.concatenate([e / esum for e in e_cols], axis=0)

    rpq = rpq_ref[0]
    samp = jnp.concatenate(
        [rpq + oa_ref[0, :, 3 * p:3 * p + 3] for p in range(P)], axis=0)
    s2 = (samp[:, 0:1] * samp[:, 0:1]
          + samp[:, 1:2] * samp[:, 1:2]
          + samp[:, 2:3] * samp[:, 2:3])
    cross = jax.lax.dot_general(
        samp.astype(jnp.bfloat16), rpkT.astype(jnp.bfloat16),
        (((1,), (0,)), ((), ())),
        preferred_element_type=jnp.float32)
    d2 = jnp.maximum(s2 + rpk2 - 2.0 * cross, 0.0)

    fi = jax.lax.broadcasted_iota(jnp.int32, (M, L), 1).astype(jnp.float32)
    acc = jnp.zeros((M, L), jnp.float32)
    ssum = jnp.zeros((M, 1), jnp.float32)
    for k in range(K):
        rowmin = jnp.min(d2, axis=-1, keepdims=True)
        cand = jnp.where(d2 == rowmin, fi, jnp.float32(L))
        idx = jnp.min(cand, axis=-1, keepdims=True)
        sel = cand == idx
        w = 1.0 / (jnp.sqrt(rowmin) + 1e-8)
        acc = jnp.where(sel, w, acc)
        ssum = ssum + w
        if k + 1 < K:
            d2 = jnp.where(sel, INF, d2)
    wmat = acc * (att4 / ssum)

    out4 = jnp.dot(wmat, val_ref[0], preferred_element_type=jnp.float32)
    o_ref[0] = (out4[0 * LB:1 * LB] + out4[1 * LB:2 * LB]
                + out4[2 * LB:3 * LB] + out4[3 * LB:4 * LB])


def _out_kernel(hd_ref, w_ref, b_ref, o_ref):
    i = pl.program_id(0)

    @pl.when(i % H == 0)
    def _init():
        o_ref[0] = jnp.broadcast_to(b_ref[...], (L, E))

    o_ref[0] += jnp.dot(hd_ref[0], w_ref[...],
                        preferred_element_type=jnp.float32)


def kernel(query_features, reference_points, W_val, b_val, W_off, b_off,
           W_att, b_att, W_out, b_out):
    qf = query_features.reshape(N * L, E)
    Wc = jnp.concatenate([W_val, W_off, W_att], axis=1)
    bc = jnp.concatenate([b_val, b_off, b_att]).reshape(1, -1)

    VALH, OAH = pl.pallas_call(
        _proj_kernel,
        out_shape=[
            jax.ShapeDtypeStruct((H, N * L, D), jnp.float32),
            jax.ShapeDtypeStruct((H, N * L, OA), jnp.float32),
        ],
    )(qf, Wc, bc)

    rpT = reference_points.transpose(0, 2, 1)
    NB = L // LB

    heads = pl.pallas_call(
        _deform_kernel,
        grid=(NH, NB),
        in_specs=[
            pl.BlockSpec((1, LB, OA), lambda i, j: (i % H, (i // H) * NB + j, 0)),
            pl.BlockSpec((1, LB, 3), lambda i, j: (i // H, j, 0)),
            pl.BlockSpec((1, 3, L), lambda i, j: (i % N, 0, 0)),
            pl.BlockSpec((1, L, D), lambda i, j: (i // N, i % N, 0)),
        ],
        out_specs=pl.BlockSpec((1, LB, D), lambda i, j: (i, j, 0)),
        out_shape=jax.ShapeDtypeStruct((NH, L, D), jnp.float32),
        compiler_params=pltpu.CompilerParams(
            dimension_semantics=("parallel", "parallel")),
    )(OAH, reference_points, rpT, VALH)

    out = pl.pallas_call(
        _out_kernel,
        grid=(NH,),
        in_specs=[
            pl.BlockSpec((1, L, D), lambda i: (i, 0, 0)),
            pl.BlockSpec((D, E), lambda i: (i % H, 0)),
            pl.BlockSpec((1, E), lambda i: (0, 0)),
        ],
        out_specs=pl.BlockSpec((1, L, E), lambda i: (i // H, 0, 0)),
        out_shape=jax.ShapeDtypeStruct((N, L, E), jnp.float32),
    )(heads, W_out, b_out.reshape(1, E))
    return out


# speedup vs baseline: 31.0458x; 1.0224x over previous
import jax
import jax.numpy as jnp
from jax.experimental import pallas as pl
from jax.experimental.pallas import tpu as pltpu

N, L, E = 2, 1024, 256
H, P, K = 8, 4, 4
D = E // H
NH = N * H
LB = 512
OA = P * 3 + P
INF = 3.0e38


def _proj_kernel(qf_ref, wc_ref, bc_ref, valh_ref, oah_ref):
    x = (jnp.dot(qf_ref[...], wc_ref[...], preferred_element_type=jnp.float32)
         + bc_ref[...])
    for h in range(H):
        valh_ref[h] = x[:, h * D:(h + 1) * D]
        oah_ref[h, :, 0:P * 3] = x[:, E + h * P * 3:E + (h + 1) * P * 3]
        oah_ref[h, :, P * 3:OA] = x[:, E + H * P * 3 + h * P:
                                    E + H * P * 3 + (h + 1) * P]


def _deform_kernel(oa_ref, rpq_ref, rpkT_ref, val_ref, o_ref):
    M = P * LB
    rpkT = rpkT_ref[0]
    rpk2 = (rpkT[0:1, :] * rpkT[0:1, :]
            + rpkT[1:2, :] * rpkT[1:2, :]
            + rpkT[2:3, :] * rpkT[2:3, :])

    a_cols = [oa_ref[0, :, P * 3 + p:P * 3 + p + 1] for p in range(P)]
    amax = jnp.maximum(jnp.maximum(a_cols[0], a_cols[1]),
                       jnp.maximum(a_cols[2], a_cols[3]))
    e_cols = [jnp.exp(a - amax) for a in a_cols]
    esum = e_cols[0] + e_cols[1] + e_cols[2] + e_cols[3]
    att4 = jnp.concatenate([e / esum for e in e_cols], axis=0)

    rpq = rpq_ref[0]
    samp = jnp.concatenate(
        [rpq + oa_ref[0, :, 3 * p:3 * p + 3] for p in range(P)], axis=0)
    s2 = (samp[:, 0:1] * samp[:, 0:1]
          + samp[:, 1:2] * samp[:, 1:2]
          + samp[:, 2:3] * samp[:, 2:3])
    cross = jax.lax.dot_general(
        samp.astype(jnp.bfloat16), rpkT.astype(jnp.bfloat16),
        (((1,), (0,)), ((), ())),
        preferred_element_type=jnp.float32)
    d2 = jnp.maximum(s2 + rpk2 - 2.0 * cross, 0.0)

    fi = jax.lax.broadcasted_iota(jnp.int32, (M, L), 1).astype(jnp.float32)
    acc = jnp.zeros((M, L), jnp.float32)
    ssum = jnp.zeros((M, 1), jnp.float32)
    for k in range(K):
        rowmin = jnp.min(d2, axis=-1, keepdims=True)
        cand = jnp.where(d2 == rowmin, fi, jnp.float32(L))
        idx = jnp.min(cand, axis=-1, keepdims=True)
        sel = cand == idx
        w = 1.0 / (jnp.sqrt(rowmin) + 1e-8)
        acc = jnp.where(sel, w, acc)
        ssum = ssum + w
        if k + 1 < K:
            d2 = jnp.where(sel, INF, d2)
    wmat = acc * (att4 / ssum)

    out4 = jnp.dot(wmat, val_ref[0], preferred_element_type=jnp.float32)
    o_ref[0] = (out4[0 * LB:1 * LB] + out4[1 * LB:2 * LB]
                + out4[2 * LB:3 * LB] + out4[3 * LB:4 * LB])


def _out_kernel(hd_ref, w_ref, b_ref, o_ref):
    i = pl.program_id(0)

    @pl.when(i % H == 0)
    def _init():
        o_ref[0] = jnp.broadcast_to(b_ref[...], (L, E))

    o_ref[0] += jnp.dot(hd_ref[0], w_ref[...],
                        preferred_element_type=jnp.float32)


def kernel(query_features, reference_points, W_val, b_val, W_off, b_off,
           W_att, b_att, W_out, b_out):
    qf = query_features.reshape(N * L, E)
    Wc = jnp.concatenate([W_val, W_off, W_att], axis=1)
    bc = jnp.concatenate([b_val, b_off, b_att]).reshape(1, -1)

    VALH, OAH = pl.pallas_call(
        _proj_kernel,
        out_shape=[
            jax.ShapeDtypeStruct((H, N * L, D), jnp.float32),
            jax.ShapeDtypeStruct((H, N * L, OA), jnp.float32),
        ],
    )(qf, Wc, bc)

    rpT = reference_points.transpose(0, 2, 1)
    NB = L // LB

    heads = pl.pallas_call(
        _deform_kernel,
        grid=(NH, NB),
        in_specs=[
            pl.BlockSpec((1, LB, OA), lambda i, j: (i % H, (i // H) * NB + j, 0)),
            pl.BlockSpec((1, LB, 3), lambda i, j: (i // H, j, 0)),
            pl.BlockSpec((1, 3, L), lambda i, j: (i % N, 0, 0)),
            pl.BlockSpec((1, L, D), lambda i, j: (i // N, i % N, 0)),
        ],
        out_specs=pl.BlockSpec((1, LB, D), lambda i, j: (i, j, 0)),
        out_shape=jax.ShapeDtypeStruct((NH, L, D), jnp.float32),
        compiler_params=pltpu.CompilerParams(
            dimension_semantics=("parallel", "parallel")),
    )(OAH, reference_points, rpT, VALH)

    out = pl.pallas_call(
        _out_kernel,
        grid=(NH,),
        in_specs=[
            pl.BlockSpec((1, L, D), lambda i: (i, 0, 0)),
            pl.BlockSpec((D, E), lambda i: (i % H, 0)),
            pl.BlockSpec((1, E), lambda i: (0, 0)),
        ],
        out_specs=pl.BlockSpec((1, L, E), lambda i: (i // H, 0, 0)),
        out_shape=jax.ShapeDtypeStruct((N, L, E), jnp.float32),
    )(heads, W_out, b_out.reshape(1, E))
    return out
